# baseline (speedup 1.0000x reference)
import functools
import numpy as np
import jax
import jax.numpy as jnp
from jax import lax
from jax.experimental import pallas as pl
from jax.experimental.pallas import tpu as pltpu
from jax.experimental.pallas import tpu_sc as plsc

_F = 16
_S = 8
_K = _F * (_F - 1)
_LN = 128
_SB = 8
_WPM = _LN // _S - 1
_WPG = 5
_NG = _WPM // _WPG
_RS = _WPG * _S + _F - _S
_BPW = 2


def _patterns(filter_size):
    r, c = np.triu_indices(_F, 1)
    rr = np.concatenate([r, c]) + (filter_size - _F)
    cc = np.concatenate([c, r]) + (filter_size - _F)
    tab = np.concatenate(
        [rr >> 3, rr & 7, cc,
         cc >> 3, rr >> 3, rr & 7, cc & 7])
    return jnp.asarray(tab.astype(np.int32))


def kernel(adj, filter_size, stride):
    B, C, D, n, _ = adj.shape
    W = (n - _F) // _S + 1
    NM = n // _LN
    NX = NM - 1
    NGRP = NM // _BPW
    try:
        fs = int(filter_size)
    except (TypeError, jax.errors.TracerIntegerConversionError):
        fs = _F
    assert W == NM * _WPM + NX
    tab_host = _patterns(fs)

    info = plsc.get_sparse_core_info()
    num_cores = info.num_cores
    assert num_cores * info.num_subcores == B * C * NGRP

    adj6 = jnp.reshape(adj, (B, C, D, n // _SB, _SB, n // _LN, _LN))
    adj6 = jnp.transpose(adj6, (0, 1, 2, 3, 5, 4, 6))
    adj6 = jnp.reshape(adj6, (B, C, D, n // _SB, (n // _LN) * _SB, _LN))

    WB = NM * _BPW
    KB = D * _K // _LN + 1

    nvx = _K // 16
    mesh = plsc.VectorSubcoreMesh(core_axis_name="c", subcore_axis_name="s")

    @functools.partial(
        pl.kernel,
        mesh=mesh,
        compiler_params=pltpu.CompilerParams(
            use_tc_tiling_on_sc=False, needs_layout_passes=False),
        out_type=jax.ShapeDtypeStruct((B, C, WB, KB, _SB, _LN), jnp.float32),
        scratch_types=[
            pltpu.VMEM((7 * _K,), jnp.int32),
            pltpu.VMEM((2, D, _NG, _RS // _SB, _SB, _RS), jnp.float32),
            pltpu.VMEM((2, D, 2, 2, _SB, _SB), jnp.float32),
            pltpu.VMEM((2, _BPW, KB, _SB, _LN), jnp.float32),
            pltpu.SemaphoreType.DMA,
            pltpu.SemaphoreType.DMA,
            pltpu.SemaphoreType.DMA,
            pltpu.SemaphoreType.DMA,
            pltpu.SemaphoreType.DMA,
            pltpu.SemaphoreType.DMA,
        ],
    )
    def run(adj_hbm, tab_hbm, out_hbm, tab, buf, bufx,
            obuf, si0, si1, sx0, sx1, so0, so1):
        wid = lax.axis_index("s") * num_cores + lax.axis_index("c")
        b = wid // (C * NGRP)
        c = (wid // NGRP) % C
        g = wid % NGRP

        def rect_src(m, d, r):
            return adj_hbm.at[
                b, c, d,
                pl.ds(m * (_LN // _SB) + r * _WPG, _RS // _SB),
                pl.ds(m * _SB, _SB),
                pl.ds(r * _WPG * _S, _RS)]

        def strip_src(m, d, h):
            return adj_hbm.at[
                b, c, d,
                pl.ds(m * (_LN // _SB) + _WPM, 2),
                pl.ds(m * _SB + h * _SB, _SB),
                pl.ds((_LN - _SB) * (1 - h), _SB)]

        def rects(p, m, sem, wait):
            for d in range(D):
                for r in range(_NG):
                    cp = pltpu.make_async_copy(
                        rect_src(m, d, r), buf.at[p, d, r], sem)
                    cp.wait() if wait else cp.start()

        def strips(p, m, sem, wait):
            for d in range(D):
                for h in range(2):
                    cp = pltpu.make_async_copy(
                        strip_src(m, d, h), bufx.at[p, d, h], sem)
                    cp.wait() if wait else cp.start()

        def select_main(p):
            def djbody(i, carry):
                d, j = i // nvx, i % nvx
                b0 = tab[pl.ds(16 * j, 16)]
                b1 = tab[pl.ds(_K + 16 * j, 16)]
                b2 = tab[pl.ds(2 * _K + 16 * j, 16)]
                vals = [
                    plsc.load_gather(
                        buf.at[p, d, r], [b0 + u, b1, b2 + _S * u])
                    for r in range(_NG) for u in range(_WPG)]
                k0 = d * _K + 16 * j
                kb, kl = k0 // _LN, k0 % _LN
                for w in range(_WPM):
                    obuf[p, w >> 3, kb, w & 7, pl.ds(kl, 16)] = vals[w]
                return carry

            lax.fori_loop(0, D * nvx, djbody, 0)

        def select_crossing(p):
            def dbody(d, carry):
                for j in range(nvx):
                    i0 = tab[pl.ds(3 * _K + 16 * j, 16)]
                    i1 = tab[pl.ds(4 * _K + 16 * j, 16)]
                    i2 = tab[pl.ds(5 * _K + 16 * j, 16)]
                    i3 = tab[pl.ds(6 * _K + 16 * j, 16)]
                    vals = plsc.load_gather(bufx.at[p, d], [i0, i1, i2, i3])
                    k0 = d * _K + 16 * j
                    obuf[p, 1, k0 // _LN, _SB - 1,
                         pl.ds(k0 % _LN, 16)] = vals
                return carry

            lax.fori_loop(0, D, dbody, 0)

        mm = (2 * g, 2 * g + 1)
        mx = (2 * g, jnp.minimum(2 * g + 1, NX - 1))
        in_sem = (si0, si1)
        x_sem = (sx0, sx1)
        out_sem = (so0, so1)

        def out_dst(p):
            return out_hbm.at[b, c, pl.ds((2 * g + p) * _BPW, _BPW)]

        rects(0, mm[0], si0, False)
        strips(0, mx[0], sx0, False)
        rects(1, mm[1], si1, False)
        strips(1, mx[1], sx1, False)
        pltpu.sync_copy(tab_hbm, tab)

        for p in range(2):
            rects(p, mm[p], in_sem[p], True)
            select_main(p)
            strips(p, mx[p], x_sem[p], True)
            select_crossing(p)
            pltpu.async_copy(obuf.at[p], out_dst(p), out_sem[p])
        for p in range(2):
            pltpu.make_async_copy(obuf.at[p], out_dst(p), out_sem[p]).wait()

    out6 = run(adj6, tab_host)
    out = jnp.transpose(out6, (0, 1, 2, 4, 3, 5))
    out = jnp.reshape(out, (B, C, WB * _SB, KB * _LN))
    return out[:, :, :W, :D * _K]


# speedup vs baseline: 1.0087x; 1.0087x over previous
import functools
import numpy as np
import jax
import jax.numpy as jnp
from jax import lax
from jax.experimental import pallas as pl
from jax.experimental.pallas import tpu as pltpu
from jax.experimental.pallas import tpu_sc as plsc

_F = 16
_S = 8
_K = _F * (_F - 1)
_LN = 128
_SB = 8
_WPM = _LN // _S - 1
_WPG = 5
_NG = _WPM // _WPG
_RS = _WPG * _S + _F - _S
_BPW = 2


def _patterns(filter_size):
    r, c = np.triu_indices(_F, 1)
    rr = np.concatenate([r, c]) + (filter_size - _F)
    cc = np.concatenate([c, r]) + (filter_size - _F)
    tab = np.concatenate(
        [rr >> 3, rr & 7, cc,
         cc >> 3, rr >> 3, rr & 7, cc & 7])
    return jnp.asarray(tab.astype(np.int32))


def kernel(adj, filter_size, stride):
    B, C, D, n, _ = adj.shape
    W = (n - _F) // _S + 1
    NM = n // _LN
    NX = NM - 1
    NGRP = NM // _BPW
    try:
        fs = int(filter_size)
    except (TypeError, jax.errors.TracerIntegerConversionError):
        fs = _F
    assert W == NM * _WPM + NX
    tab_host = _patterns(fs)

    info = plsc.get_sparse_core_info()
    num_cores = info.num_cores
    assert num_cores * info.num_subcores == B * C * NGRP

    adj6 = jnp.reshape(adj, (B, C, D, n // _SB, _SB, n // _LN, _LN))
    adj6 = jnp.transpose(adj6, (0, 1, 2, 3, 5, 4, 6))
    adj6 = jnp.reshape(adj6, (B, C, D, n // _SB, (n // _LN) * _SB, _LN))

    WB = NM * _BPW
    KB = D * _K // _LN + 1

    nvx = _K // 16
    mesh = plsc.VectorSubcoreMesh(core_axis_name="c", subcore_axis_name="s")

    @functools.partial(
        pl.kernel,
        mesh=mesh,
        compiler_params=pltpu.CompilerParams(
            use_tc_tiling_on_sc=False, needs_layout_passes=False),
        out_type=jax.ShapeDtypeStruct((B, C, WB, KB, _SB, _LN), jnp.float32),
        scratch_types=[
            pltpu.VMEM((7 * _K,), jnp.int32),
            pltpu.VMEM((2, D, _NG, _RS // _SB, _SB, _RS), jnp.float32),
            pltpu.VMEM((2, D, 2, 2, _SB, _SB), jnp.float32),
            pltpu.VMEM((2, _BPW, KB, _SB, _LN), jnp.float32),
            pltpu.SemaphoreType.DMA,
            pltpu.SemaphoreType.DMA,
            pltpu.SemaphoreType.DMA,
            pltpu.SemaphoreType.DMA,
            pltpu.SemaphoreType.DMA,
            pltpu.SemaphoreType.DMA,
        ],
    )
    def run(adj_hbm, tab_hbm, out_hbm, tab, buf, bufx,
            obuf, si0, si1, sx0, sx1, so0, so1):
        wid = lax.axis_index("s") * num_cores + lax.axis_index("c")
        b = wid // (C * NGRP)
        c = (wid // NGRP) % C
        g = wid % NGRP
        pltpu.sync_copy(tab_hbm, tab)

        def rect_src(m, d, r):
            return adj_hbm.at[
                b, c, d,
                pl.ds(m * (_LN // _SB) + r * _WPG, _RS // _SB),
                pl.ds(m * _SB, _SB),
                pl.ds(r * _WPG * _S, _RS)]

        def strip_src(m, d, h):
            return adj_hbm.at[
                b, c, d,
                pl.ds(m * (_LN // _SB) + _WPM, 2),
                pl.ds(m * _SB + h * _SB, _SB),
                pl.ds((_LN - _SB) * (1 - h), _SB)]

        def rects(p, m, sem, wait):
            for d in range(D):
                for r in range(_NG):
                    cp = pltpu.make_async_copy(
                        rect_src(m, d, r), buf.at[p, d, r], sem)
                    cp.wait() if wait else cp.start()

        def strips(p, m, sem, wait):
            for d in range(D):
                for h in range(2):
                    cp = pltpu.make_async_copy(
                        strip_src(m, d, h), bufx.at[p, d, h], sem)
                    cp.wait() if wait else cp.start()

        def select_main(p):
            def djbody(i, carry):
                d, j = i // nvx, i % nvx
                b0 = tab[pl.ds(16 * j, 16)]
                b1 = tab[pl.ds(_K + 16 * j, 16)]
                b2 = tab[pl.ds(2 * _K + 16 * j, 16)]
                vals = [
                    plsc.load_gather(
                        buf.at[p, d, r], [b0 + u, b1, b2 + _S * u])
                    for r in range(_NG) for u in range(_WPG)]
                k0 = d * _K + 16 * j
                kb, kl = k0 // _LN, k0 % _LN
                for w in range(_WPM):
                    obuf[p, w >> 3, kb, w & 7, pl.ds(kl, 16)] = vals[w]
                return carry

            lax.fori_loop(0, D * nvx, djbody, 0)

        def select_crossing(p):
            def dbody(d, carry):
                for j in range(nvx):
                    i0 = tab[pl.ds(3 * _K + 16 * j, 16)]
                    i1 = tab[pl.ds(4 * _K + 16 * j, 16)]
                    i2 = tab[pl.ds(5 * _K + 16 * j, 16)]
                    i3 = tab[pl.ds(6 * _K + 16 * j, 16)]
                    vals = plsc.load_gather(bufx.at[p, d], [i0, i1, i2, i3])
                    k0 = d * _K + 16 * j
                    obuf[p, 1, k0 // _LN, _SB - 1,
                         pl.ds(k0 % _LN, 16)] = vals
                return carry

            lax.fori_loop(0, D, dbody, 0)

        mm = (2 * g, 2 * g + 1)
        mx = (2 * g, jnp.minimum(2 * g + 1, NX - 1))
        in_sem = (si0, si1)
        x_sem = (sx0, sx1)
        out_sem = (so0, so1)

        def out_dst(p):
            return out_hbm.at[b, c, pl.ds((2 * g + p) * _BPW, _BPW)]

        rects(0, mm[0], si0, False)
        strips(0, mx[0], sx0, False)
        rects(1, mm[1], si1, False)
        strips(1, mx[1], sx1, False)

        for p in range(2):
            rects(p, mm[p], in_sem[p], True)
            select_main(p)
            strips(p, mx[p], x_sem[p], True)
            select_crossing(p)
            pltpu.async_copy(obuf.at[p], out_dst(p), out_sem[p])
        for p in range(2):
            pltpu.make_async_copy(obuf.at[p], out_dst(p), out_sem[p]).wait()

    out6 = run(adj6, tab_host)
    out = jnp.transpose(out6, (0, 1, 2, 4, 3, 5))
    out = jnp.reshape(out, (B, C, WB * _SB, KB * _LN))
    return out[:, :, :W, :D * _K]
